# baseline (speedup 1.0000x reference)
import functools
import jax
import jax.numpy as jnp
from jax import lax
from jax.experimental import pallas as pl
from jax.experimental.pallas import tpu as pltpu
from jax.experimental.pallas import tpu_sc as plsc

NC = 2
NS = 16
NW = NC * NS
L = 16
B = 64
SB = 8
NBUF = 4
DW = 8
N_PAD = 10240


def _leaky(t):
    return jnp.where(t >= 0, t, 0.2 * t)


def _track_shift(i, grid, es, ed, cv_ref, acc_ref):
    m_es = jnp.max(es)
    m_ed = jnp.max(ed)
    first = i == 0
    acc_ref[0, :] = jnp.where(first, jnp.full((128,), m_es),
                              jnp.maximum(acc_ref[0, :], m_es))
    acc_ref[1, :] = jnp.where(first, jnp.full((128,), m_ed),
                              jnp.maximum(acc_ref[1, :], m_ed))

    @pl.when(i == grid - 1)
    def _():
        cv_ref[...] = _leaky(acc_ref[0, :] + acc_ref[1, :])


def _k1_body(grid, x_ref, w_ref, asrc_ref, adst_ref,
             h_ref, es_ref, ed_ref, cv_ref, acc_ref):
    i = pl.program_id(0)
    h = jnp.dot(x_ref[...], w_ref[...], preferred_element_type=jnp.float32)
    h_ref[...] = h
    es = jnp.sum(h * asrc_ref[...][None, :], axis=1)
    ed = jnp.sum(h * adst_ref[...][None, :], axis=1)
    es_ref[...] = es
    ed_ref[...] = ed
    _track_shift(i, grid, es, ed, cv_ref, acc_ref)


def _dense1(x_pad, W, a_src, a_dst):
    d_in = x_pad.shape[1]
    hid = W.shape[1]
    R = 512
    grid = N_PAD // R
    return pl.pallas_call(
        functools.partial(_k1_body, grid),
        grid=(grid,),
        in_specs=[
            pl.BlockSpec((R, d_in), lambda i: (i, 0)),
            pl.BlockSpec((d_in, hid), lambda i: (0, 0)),
            pl.BlockSpec((hid,), lambda i: (0,)),
            pl.BlockSpec((hid,), lambda i: (0,)),
        ],
        out_specs=[
            pl.BlockSpec((R, hid), lambda i: (i, 0)),
            pl.BlockSpec((R,), lambda i: (i,)),
            pl.BlockSpec((R,), lambda i: (i,)),
            pl.BlockSpec((128,), lambda i: (0,)),
        ],
        out_shape=[
            jax.ShapeDtypeStruct((N_PAD, hid), jnp.float32),
            jax.ShapeDtypeStruct((N_PAD,), jnp.float32),
            jax.ShapeDtypeStruct((N_PAD,), jnp.float32),
            jax.ShapeDtypeStruct((128,), jnp.float32),
        ],
        scratch_shapes=[pltpu.VMEM((2, 128), jnp.float32)],
    )(x_pad, W, a_src, a_dst)


def _sc_agg_body(nb, W,
                 h_hbm, es_hbm, ed_hbm, idx_hbm, cv_hbm,
                 out_hbm, wout_hbm,
                 idxb, rows, wbuf, esg, edg, cbuf,
                 out_acc, wacc, es_sh, ed_sh, sem_g, sem_e, sem_s):
    cid = lax.axis_index("c")
    sid = lax.axis_index("s")
    wid = cid * NS + sid
    stripe = N_PAD // NS
    soff = sid * stripe
    zi16 = jnp.zeros((L,), jnp.int32)
    iota16 = lax.iota(jnp.int32, L)

    pltpu.sync_copy(cv_hbm.at[pl.ds(0, L)], cbuf)
    pltpu.sync_copy(es_hbm.at[pl.ds(soff, stripe)],
                    es_sh.at[pl.ds(soff, stripe)])
    pltpu.sync_copy(ed_hbm.at[pl.ds(soff, stripe)],
                    ed_sh.at[pl.ds(soff, stripe)])

    zero = jnp.zeros((L,), jnp.float32)

    def _zrow(j, _):
        for k in range(W // L):
            rows[0, j, pl.ds(k * L, L)] = zero
        return 0

    lax.fori_loop(0, B, _zrow, 0)
    zf = jnp.zeros((L,), jnp.float32)
    for q in range(NBUF):
        for g in range(B // L):
            for c in range(DW):
                plsc.store_scatter(wbuf.at[q],
                                   [g * L + iota16, jnp.full((L,), c,
                                                             jnp.int32)], zf)
    for k in range(stripe // B):
        pltpu.sync_copy(rows.at[0], out_acc.at[pl.ds(soff + k * B, B)])
        pltpu.sync_copy(wbuf.at[0], wacc.at[pl.ds(soff + k * B, B)])
    plsc.subcore_barrier()

    cv = cbuf[pl.ds(0, L)]

    def _fire_gath(j):
        q = j % NBUF
        dr = pltpu.async_copy(h_hbm.at[idxb.at[2 * j]], rows.at[q], sem_g)
        de = pltpu.async_copy(es_sh.at[idxb.at[2 * j]], esg.at[q], sem_e)
        dd = pltpu.async_copy(ed_sh.at[idxb.at[2 * j + 1]], edg.at[q], sem_e)
        return dr, de, dd

    def _compute(j):
        q = j % NBUF
        for g in range(B // L):
            sv = esg.at[q][pl.ds(g * L, L)]
            dv = edg.at[q][pl.ds(g * L, L)]
            ee = jnp.exp(_leaky(sv + dv) - cv)
            plsc.store_scatter(wbuf.at[q], [g * L + iota16, zi16], ee)

        def _edge(jj, _):
            for u in range(8):
                e = jj * 8 + u
                wv = plsc.load_gather(
                    wbuf.at[q], [jnp.full((L,), e, jnp.int32), zi16])
                for k in range(W // L):
                    rows[q, e, pl.ds(k * L, L)] = (
                        rows[q, e, pl.ds(k * L, L)] * wv)
            return 0

        lax.fori_loop(0, B // 8, _edge, 0)

    def _fire_scat(j):
        q = j % NBUF
        s1 = pltpu.async_copy(rows.at[q], out_acc.at[idxb.at[2 * j + 1]],
                              sem_s, add=True)
        s2 = pltpu.async_copy(wbuf.at[q], wacc.at[idxb.at[2 * j + 1]],
                              sem_s, add=True)
        return s1, s2

    def _super(i, _):
        pltpu.sync_copy(idx_hbm.at[wid].at[i], idxb)
        gd = {0: _fire_gath(0), 1: _fire_gath(1)}
        sd = {}
        for j in range(SB):
            for d in gd[j]:
                d.wait()
            _compute(j)
            sd[j] = _fire_scat(j)
            if j >= 2:
                for d in sd[j - 2]:
                    d.wait()
            if j + 2 < SB:
                gd[j + 2] = _fire_gath(j + 2)
        for d in sd[SB - 2] + sd[SB - 1]:
            d.wait()
        return 0

    lax.fori_loop(0, nb // SB, _super, 0)
    plsc.subcore_barrier()

    for k in range(stripe // B):
        off = soff + k * B
        pltpu.sync_copy(out_acc.at[pl.ds(off, B)],
                        out_hbm.at[cid].at[pl.ds(off, B)])
        pltpu.sync_copy(wacc.at[pl.ds(off, B)],
                        wout_hbm.at[cid].at[pl.ds(off, B)])


def _sc_aggregate(h_pad, es, ed, idx4, cv):
    W = h_pad.shape[1]
    nb = idx4.shape[1] * SB
    mesh = plsc.VectorSubcoreMesh(core_axis_name="c", subcore_axis_name="s")
    body = functools.partial(_sc_agg_body, nb, W)
    return pl.kernel(
        body,
        out_type=[
            jax.ShapeDtypeStruct((NC, N_PAD, W), jnp.float32),
            jax.ShapeDtypeStruct((NC, N_PAD, DW), jnp.float32),
        ],
        mesh=mesh,
        compiler_params=pltpu.CompilerParams(needs_layout_passes=False,
                                             use_tc_tiling_on_sc=False),
        scratch_types=[
            pltpu.VMEM((2 * SB, B), jnp.int32),
            pltpu.VMEM((NBUF, B, W), jnp.float32),
            pltpu.VMEM((NBUF, B, DW), jnp.float32),
            pltpu.VMEM((NBUF, B), jnp.float32),
            pltpu.VMEM((NBUF, B), jnp.float32),
            pltpu.VMEM((L,), jnp.float32),
            pltpu.VMEM_SHARED((N_PAD, W), jnp.float32),
            pltpu.VMEM_SHARED((N_PAD, DW), jnp.float32),
            pltpu.VMEM_SHARED((N_PAD,), jnp.float32),
            pltpu.VMEM_SHARED((N_PAD,), jnp.float32),
            pltpu.SemaphoreType.DMA,
            pltpu.SemaphoreType.DMA,
            pltpu.SemaphoreType.DMA,
        ],
    )(h_pad, es, ed, idx4, cv)


def _k2_body(grid, s_ref, d_ref, b1_ref, w2_ref, a2s_ref, a2d_ref,
             h2_ref, es_ref, ed_ref, cv_ref, acc_ref):
    i = pl.program_id(0)
    num = s_ref[0] + s_ref[1]
    den = (d_ref[0] + d_ref[1])[:, 0:1]
    den = jnp.where(den > 0, den, 1.0)
    h1 = jax.nn.relu(num / den + b1_ref[...][None, :])
    h2 = jnp.dot(h1, w2_ref[...], preferred_element_type=jnp.float32)
    h2_ref[...] = h2
    es = jnp.sum(h2 * a2s_ref[...][None, :], axis=1)
    ed = jnp.sum(h2 * a2d_ref[...][None, :], axis=1)
    es_ref[...] = es
    ed_ref[...] = ed
    _track_shift(i, grid, es, ed, cv_ref, acc_ref)


def _dense2(S1, D1, b1, W2p, a2s_p, a2d_p):
    hid = W2p.shape[0]
    lw = W2p.shape[1]
    R = 512
    grid = N_PAD // R
    return pl.pallas_call(
        functools.partial(_k2_body, grid),
        grid=(grid,),
        in_specs=[
            pl.BlockSpec((NC, R, hid), lambda i: (0, i, 0)),
            pl.BlockSpec((NC, R, DW), lambda i: (0, i, 0)),
            pl.BlockSpec((hid,), lambda i: (0,)),
            pl.BlockSpec((hid, lw), lambda i: (0, 0)),
            pl.BlockSpec((lw,), lambda i: (0,)),
            pl.BlockSpec((lw,), lambda i: (0,)),
        ],
        out_specs=[
            pl.BlockSpec((R, lw), lambda i: (i, 0)),
            pl.BlockSpec((R,), lambda i: (i,)),
            pl.BlockSpec((R,), lambda i: (i,)),
            pl.BlockSpec((128,), lambda i: (0,)),
        ],
        out_shape=[
            jax.ShapeDtypeStruct((N_PAD, lw), jnp.float32),
            jax.ShapeDtypeStruct((N_PAD,), jnp.float32),
            jax.ShapeDtypeStruct((N_PAD,), jnp.float32),
            jax.ShapeDtypeStruct((128,), jnp.float32),
        ],
        scratch_shapes=[pltpu.VMEM((2, 128), jnp.float32)],
    )(S1, D1, b1, W2p, a2s_p, a2d_p)


def _k3_body(n_label, s_ref, d_ref, b2_ref, o_ref):
    num = (s_ref[0] + s_ref[1])[:, :n_label]
    den = (d_ref[0] + d_ref[1])[:, 0:1]
    den = jnp.where(den > 0, den, 1.0)
    logits = num / den + b2_ref[...][None, :]
    m = jnp.max(logits, axis=1, keepdims=True)
    z = logits - m
    o_ref[...] = z - jnp.log(jnp.sum(jnp.exp(z), axis=1, keepdims=True))


def _dense3(S2, D2, b2, n, n_label):
    lw = S2.shape[2]
    R = 512
    grid = N_PAD // R
    return pl.pallas_call(
        functools.partial(_k3_body, n_label),
        grid=(grid,),
        in_specs=[
            pl.BlockSpec((NC, R, lw), lambda i: (0, i, 0)),
            pl.BlockSpec((NC, R, DW), lambda i: (0, i, 0)),
            pl.BlockSpec((n_label,), lambda i: (0,)),
        ],
        out_specs=pl.BlockSpec((R, n_label), lambda i: (i, 0)),
        out_shape=jax.ShapeDtypeStruct((n, n_label), jnp.float32),
    )(S2, D2, b2)


@jax.jit
def kernel(x, W1, a1_src, a1_dst, b1, W2, a2_src, a2_dst, b2, edge_index):
    n, d_in = x.shape
    hid = W1.shape[1]
    n_label = W2.shape[1]
    e = edge_index.shape[1]
    e_tot = e + n

    nb = -(-e_tot // (NW * B))
    nb += (-nb) % SB
    e_pad = NW * nb * B
    loop = jnp.arange(n, dtype=jnp.int32)
    src = jnp.concatenate([edge_index[0].astype(jnp.int32), loop])
    dst = jnp.concatenate([edge_index[1].astype(jnp.int32), loop])
    pad_n = jnp.full((e_pad - e_tot,), n, jnp.int32)
    src4 = jnp.concatenate([src, pad_n]).reshape(NW, nb // SB, SB, 1, B)
    dst4 = jnp.concatenate([dst, pad_n]).reshape(NW, nb // SB, SB, 1, B)
    idx4 = jnp.concatenate([src4, dst4], axis=3).reshape(
        NW, nb // SB, 2 * SB, B)
    x_pad = jnp.zeros((N_PAD, d_in), jnp.float32).at[:n].set(x)

    h1, es1, ed1, cv1 = _dense1(x_pad, W1, a1_src, a1_dst)
    S1, D1 = _sc_aggregate(h1, es1, ed1, idx4, cv1)

    lw = 48
    W2p = jnp.zeros((hid, lw), jnp.float32).at[:, :n_label].set(W2)
    a2s_p = jnp.zeros((lw,), jnp.float32).at[:n_label].set(a2_src)
    a2d_p = jnp.zeros((lw,), jnp.float32).at[:n_label].set(a2_dst)
    h2, es2, ed2, cv2 = _dense2(S1, D1, b1, W2p, a2s_p, a2d_p)

    S2, D2 = _sc_aggregate(h2, es2, ed2, idx4, cv2)
    return _dense3(S2, D2, b2, n, n_label)
